# Initial kernel scaffold; baseline (speedup 1.0000x reference)
import jax
import jax.numpy as jnp
from jax.experimental import pallas as pl


def kernel(edge_index, r, atom_features, distances, graph_ids, af_table, W_atom, b_atom, W_dist, b_dist, ln_g, ln_b, W_edge, b_edge, W_pre0, b_pre0, W_post0, b_post0, W_pre1, b_pre1, W_post1, b_post1, W_out, b_out):
    raise NotImplementedError("write your pallas kernel here")


# speedup vs baseline: 2.1345x; 2.1345x over previous
import functools
import math

import jax
import jax.numpy as jnp
from jax import lax
from jax.experimental import pallas as pl
from jax.experimental.pallas import tpu as pltpu
from jax.experimental.pallas import tpu_sc as plsc

N = 10000
E = 160000
D = 128
EF = 40
MAXN = 12
NG = 100
NATOM = 100
AENC = 200
DELTA = float(math.log(MAXN + 1.0))

NT = 32
RPV = 160
VT = 64
RPT = 2 * RPV
NPAD = VT * RPV
SCRH = 16 * RPV
FB = 512
CAP = 313 * FB
KCH = 2048
NCHP = 79
EPADP = NCHP * KCH
SENT = 1 << 24


def _prep_weights(af_table, W_atom, b_atom, W_edge, b_edge, W_pre0, b_pre0,
                  W_pre1, b_pre1):
    def body(af_ref, wa_ref, ba_ref, we_ref, be_ref, wp0_ref, bp0_ref,
             wp1_ref, bp1_ref, t2_ref, wc0_ref, cb0_ref, wc1_ref, cb1_ref):
        t2_ref[...] = jnp.dot(af_ref[...], wa_ref[...],
                              preferred_element_type=jnp.float32) + ba_ref[...]
        wpc0 = wp0_ref[2 * D:3 * D, :]
        wpc1 = wp1_ref[2 * D:3 * D, :]
        wc0_ref[...] = jnp.dot(we_ref[...], wpc0,
                               preferred_element_type=jnp.float32)
        wc1_ref[...] = jnp.dot(we_ref[...], wpc1,
                               preferred_element_type=jnp.float32)
        cb0_ref[...] = jnp.dot(be_ref[...], wpc0,
                               preferred_element_type=jnp.float32) + bp0_ref[...]
        cb1_ref[...] = jnp.dot(be_ref[...], wpc1,
                               preferred_element_type=jnp.float32) + bp1_ref[...]

    return pl.pallas_call(
        body,
        out_shape=(
            jax.ShapeDtypeStruct((NATOM, D), jnp.float32),
            jax.ShapeDtypeStruct((EF, D), jnp.float32),
            jax.ShapeDtypeStruct((1, D), jnp.float32),
            jax.ShapeDtypeStruct((EF, D), jnp.float32),
            jax.ShapeDtypeStruct((1, D), jnp.float32),
        ),
    )(af_table, W_atom, b_atom.reshape(1, D), W_edge, b_edge.reshape(1, D),
      W_pre0, b_pre0.reshape(1, D), W_pre1, b_pre1.reshape(1, D))


def _embed(af2d, distances, table2, W_dist, b_dist, ln_g, ln_b, W_pre0):
    BN = 1000

    def body(af_ref, di_ref, t2_ref, wd_ref, bd_ref, g_ref, b_ref, wp_ref,
             h_ref, a_ref, bb_ref):
        onehot = jnp.where(
            af_ref[...] == lax.broadcasted_iota(jnp.int32, (BN, NATOM), 1),
            1.0, 0.0).astype(jnp.float32)
        hpre = (jnp.dot(onehot, t2_ref[...], preferred_element_type=jnp.float32)
                + jnp.dot(di_ref[...], wd_ref[...],
                          preferred_element_type=jnp.float32) + bd_ref[...])
        mu = jnp.mean(hpre, axis=-1, keepdims=True)
        var = jnp.mean((hpre - mu) ** 2, axis=-1, keepdims=True)
        h = (hpre - mu) * lax.rsqrt(var + 1e-5) * g_ref[...] + b_ref[...]
        h_ref[...] = h
        a_ref[...] = jnp.dot(h, wp_ref[0:D, :],
                             preferred_element_type=jnp.float32)
        bb_ref[...] = jnp.dot(h, wp_ref[D:2 * D, :],
                              preferred_element_type=jnp.float32)

    return pl.pallas_call(
        body,
        grid=(N // BN,),
        in_specs=[
            pl.BlockSpec((BN, 1), lambda i: (i, 0)),
            pl.BlockSpec((BN, MAXN), lambda i: (i, 0)),
            pl.BlockSpec((NATOM, D), lambda i: (0, 0)),
            pl.BlockSpec((MAXN, D), lambda i: (0, 0)),
            pl.BlockSpec((1, D), lambda i: (0, 0)),
            pl.BlockSpec((1, D), lambda i: (0, 0)),
            pl.BlockSpec((1, D), lambda i: (0, 0)),
            pl.BlockSpec((3 * D, D), lambda i: (0, 0)),
        ],
        out_specs=[
            pl.BlockSpec((BN, D), lambda i: (i, 0)),
            pl.BlockSpec((BN, D), lambda i: (i, 0)),
            pl.BlockSpec((BN, D), lambda i: (i, 0)),
        ],
        out_shape=[
            jax.ShapeDtypeStruct((N, D), jnp.float32),
            jax.ShapeDtypeStruct((N, D), jnp.float32),
            jax.ShapeDtypeStruct((N, D), jnp.float32),
        ],
    )(af2d, distances, table2, W_dist, b_dist.reshape(1, D),
      ln_g.reshape(1, D), ln_b.reshape(1, D), W_pre0)


def _edge_c(r, wc0, cb0, wc1, cb1):
    BE = 2000
    gamma = float((EF - 1) ** 2)

    def body(r_ref, wc0_ref, cb0_ref, wc1_ref, cb1_ref, c0_ref, c1_ref):
        rr = r_ref[...]
        d2 = jnp.sum(rr * rr, axis=1, keepdims=True)
        x = lax.rsqrt(d2)
        centers = lax.broadcasted_iota(
            jnp.int32, (BE, EF), 1).astype(jnp.float32) * (1.0 / (EF - 1))
        de = jnp.exp(-gamma * (x - centers) ** 2)
        c0_ref[...] = jnp.dot(de, wc0_ref[...],
                              preferred_element_type=jnp.float32) + cb0_ref[...]
        c1_ref[...] = jnp.dot(de, wc1_ref[...],
                              preferred_element_type=jnp.float32) + cb1_ref[...]

    return pl.pallas_call(
        body,
        grid=(E // BE,),
        in_specs=[
            pl.BlockSpec((BE, 3), lambda i: (i, 0)),
            pl.BlockSpec((EF, D), lambda i: (0, 0)),
            pl.BlockSpec((1, D), lambda i: (0, 0)),
            pl.BlockSpec((EF, D), lambda i: (0, 0)),
            pl.BlockSpec((1, D), lambda i: (0, 0)),
        ],
        out_specs=[
            pl.BlockSpec((BE, D), lambda i: (i, 0)),
            pl.BlockSpec((BE, D), lambda i: (i, 0)),
        ],
        out_shape=[
            jax.ShapeDtypeStruct((E, D), jnp.float32),
            jax.ShapeDtypeStruct((E, D), jnp.float32),
        ],
    )(r, wc0, cb0, wc1, cb1)


def _sc_prep(dst_p, src_p):
    mesh = plsc.VectorSubcoreMesh(core_axis_name="c", subcore_axis_name="s")

    @functools.partial(
        pl.kernel,
        out_type=(
            pltpu.HBM((VT * CAP,), jnp.int32),
            pltpu.HBM((VT * CAP,), jnp.int32),
            pltpu.HBM((VT * CAP,), jnp.int32),
            pltpu.HBM((VT * 8,), jnp.int32),
            pltpu.HBM((NT * RPT * 16,), jnp.float32),
        ),
        mesh=mesh,
        compiler_params=pltpu.CompilerParams(needs_layout_passes=False),
        scratch_types=[
            pltpu.VMEM((KCH,), jnp.int32),
            pltpu.VMEM((KCH,), jnp.int32),
            pltpu.VMEM((3072,), jnp.int32),
            pltpu.VMEM((3072,), jnp.int32),
            pltpu.VMEM((3072,), jnp.int32),
            pltpu.VMEM((3072,), jnp.int32),
            pltpu.VMEM((3072,), jnp.int32),
            pltpu.VMEM((3072,), jnp.int32),
            pltpu.VMEM((RPT * 16,), jnp.float32),
            pltpu.VMEM((16,), jnp.int32),
        ],
    )
    def k(dst_h, src_h, cd_h, cs_h, ce_h, nb_h, deg_h,
          dstb, srcb, cbd0, cbs0, cbe0, cbd1, cbs1, cbe1, degacc, stg):
        c = lax.axis_index("c")
        s = lax.axis_index("s")
        tid = c * 16 + s
        lo = tid * RPT
        mid = lo + RPV
        hi = lo + RPT
        iota16 = lax.broadcasted_iota(jnp.int32, (16,), 0)
        z16 = jnp.zeros((16,), jnp.float32)
        one16 = jnp.full((16,), 1.0, jnp.float32)
        sentv = jnp.full((16,), SENT, jnp.int32)

        def zero_deg(i, carry):
            degacc[pl.ds(i * 16, 16)] = z16
            return carry

        _ = lax.fori_loop(0, RPT, zero_deg, 0)

        def make_flush(cbd, cbs, cbe, vt):
            tcap = vt * CAP

            def flush_body(st):
                cnt, nb = st
                pltpu.sync_copy(cbd.at[pl.ds(0, FB)],
                                cd_h.at[pl.ds(tcap + nb * FB, FB)])
                pltpu.sync_copy(cbs.at[pl.ds(0, FB)],
                                cs_h.at[pl.ds(tcap + nb * FB, FB)])
                pltpu.sync_copy(cbe.at[pl.ds(0, FB)],
                                ce_h.at[pl.ds(tcap + nb * FB, FB)])

                def deg_group(g, carry):
                    dv = cbd[pl.ds(g * 16, 16)]
                    for lane in range(16):
                        dg = dv[lane]

                        @pl.when(dg < hi)
                        def _upd():
                            dl = dg - lo
                            dvec = degacc[pl.ds(dl * 16, 16)]
                            degacc[pl.ds(dl * 16, 16)] = dvec + one16
                    return carry

                _ = lax.fori_loop(0, FB // 16, deg_group, 0)
                nmv = cnt - FB
                ngrp = (nmv + 15) // 16

                def move(g, carry):
                    cbd[pl.ds(g * 16, 16)] = cbd[pl.ds(FB + g * 16, 16)]
                    cbs[pl.ds(g * 16, 16)] = cbs[pl.ds(FB + g * 16, 16)]
                    cbe[pl.ds(g * 16, 16)] = cbe[pl.ds(FB + g * 16, 16)]
                    return carry

                _ = lax.fori_loop(0, ngrp, move, 0)
                return (cnt - FB, nb + 1)

            return flush_body

        flush0 = make_flush(cbd0, cbs0, cbe0, 2 * tid)
        flush1 = make_flush(cbd1, cbs1, cbe1, 2 * tid + 1)

        def chunk_body(ci, st):
            cnt0, nb0, cnt1, nb1 = st
            pltpu.sync_copy(dst_h.at[pl.ds(ci * KCH, KCH)], dstb)
            pltpu.sync_copy(src_h.at[pl.ds(ci * KCH, KCH)], srcb)
            ebase = ci * KCH

            def scan_vec(v, st2):
                c0, c1 = st2
                d = dstb[pl.ds(v * 16, 16)]
                sv = srcb[pl.ds(v * 16, 16)]
                eid = ebase + v * 16 + iota16
                valid = eid < E
                m0 = (d >= lo) & (d < mid) & valid
                m1 = (d >= mid) & (d < hi) & valid
                pos0 = plsc.cumsum(jnp.where(m0, jnp.int32(1), jnp.int32(0)))
                pos1 = plsc.cumsum(jnp.where(m1, jnp.int32(1), jnp.int32(0)))
                tgt0 = pos0 - 1 + c0
                tgt1 = pos1 - 1 + c1
                plsc.store_scatter(cbd0, [tgt0], d, mask=m0)
                plsc.store_scatter(cbs0, [tgt0], sv, mask=m0)
                plsc.store_scatter(cbe0, [tgt0], eid, mask=m0)
                plsc.store_scatter(cbd1, [tgt1], d, mask=m1)
                plsc.store_scatter(cbs1, [tgt1], sv, mask=m1)
                plsc.store_scatter(cbe1, [tgt1], eid, mask=m1)
                return (c0 + pos0[15], c1 + pos1[15])

            cnt0, cnt1 = lax.fori_loop(0, KCH // 16, scan_vec, (cnt0, cnt1))
            cnt0, nb0 = lax.while_loop(lambda st2: st2[0] >= FB, flush0,
                                       (cnt0, nb0))
            cnt1, nb1 = lax.while_loop(lambda st2: st2[0] >= FB, flush1,
                                       (cnt1, nb1))
            return (cnt0, nb0, cnt1, nb1)

        zero = jnp.int32(0)
        cnt0, nb0, cnt1, nb1 = lax.fori_loop(
            0, NCHP, chunk_body, (zero, zero, zero, zero))

        def pad(kk, carry):
            cbd0[pl.ds(cnt0 + kk * 16, 16)] = sentv
            cbs0[pl.ds(cnt0 + kk * 16, 16)] = sentv
            cbe0[pl.ds(cnt0 + kk * 16, 16)] = sentv
            cbd1[pl.ds(cnt1 + kk * 16, 16)] = sentv
            cbs1[pl.ds(cnt1 + kk * 16, 16)] = sentv
            cbe1[pl.ds(cnt1 + kk * 16, 16)] = sentv
            return carry

        _ = lax.fori_loop(0, FB // 16, pad, 0)
        _, nb0 = flush0((jnp.int32(FB), nb0))
        _, nb1 = flush1((jnp.int32(FB), nb1))

        stg[pl.ds(0, 16)] = jnp.where(iota16 == 0, nb0, 0)
        pltpu.sync_copy(stg.at[pl.ds(0, 8)], nb_h.at[pl.ds(2 * tid * 8, 8)])
        stg[pl.ds(0, 16)] = jnp.where(iota16 == 0, nb1, 0)
        pltpu.sync_copy(stg.at[pl.ds(0, 8)],
                        nb_h.at[pl.ds((2 * tid + 1) * 8, 8)])
        pltpu.sync_copy(degacc, deg_h.at[pl.ds(tid * RPT * 16, RPT * 16)])

    return k(dst_p, src_p)


def _sc_layer(A, B, C, cd, cs, ce, nb_arr):
    mesh = plsc.VectorSubcoreMesh(core_axis_name="c", subcore_axis_name="s")

    @functools.partial(
        pl.kernel,
        out_type=(
            pltpu.HBM((NPAD, D), jnp.float32),
            pltpu.HBM((NPAD * D,), jnp.float32),
            pltpu.HBM((NPAD * D,), jnp.float32),
            pltpu.HBM((NPAD * D,), jnp.float32),
        ),
        mesh=mesh,
        compiler_params=pltpu.CompilerParams(needs_layout_passes=False),
        scratch_types=[
            pltpu.VMEM((FB,), jnp.int32),
            pltpu.VMEM((FB,), jnp.int32),
            pltpu.VMEM((FB,), jnp.int32),
            pltpu.VMEM((1, 64), jnp.int32),
            pltpu.VMEM((1, 64), jnp.int32),
            pltpu.VMEM((1, 64), jnp.int32),
            pltpu.VMEM((1, 64), jnp.int32),
            pltpu.VMEM((64, D), jnp.float32),
            pltpu.VMEM((64, D), jnp.float32),
            pltpu.VMEM((64, D), jnp.float32),
            pltpu.VMEM((RPV * D,), jnp.float32),
            pltpu.VMEM((RPV * D,), jnp.float32),
            pltpu.VMEM((RPV * D,), jnp.float32),
            pltpu.VMEM((32, D), jnp.float32),
            pltpu.VMEM((16,), jnp.int32),
            pltpu.VMEM_SHARED((SCRH + 1, D), jnp.float32),
            pltpu.SemaphoreType.DMA,
            pltpu.SemaphoreType.DMA,
        ],
    )
    def k(a_h, b_h, c_h, cd_h, cs_h, ce_h, nb_h,
          sum_h, sq_h, mn_h, mx_h,
          cbd, cbs, cbe, sstg, dstg, estg, lstg,
          bufa, bufb, bufc, sqacc, mnacc, mxacc, zbuf, cntb,
          spm_sum, semg, sems):
        c = lax.axis_index("c")
        s = lax.axis_index("s")
        nmax = jnp.int32(N - 1)
        emax = jnp.int32(E - 1)
        trash = jnp.int32(SCRH)

        ninf = jnp.full((16,), -jnp.inf, jnp.float32)
        pinf = jnp.full((16,), jnp.inf, jnp.float32)
        z16 = jnp.zeros((16,), jnp.float32)

        def zrow(i, carry):
            for col in range(0, D, 16):
                zbuf[i, pl.ds(col, 16)] = z16
            return carry

        _ = lax.fori_loop(0, 32, zrow, 0)

        def pass_body(h, carry):
            vt = c * 32 + h * 16 + s
            lov = vt * RPV
            hiv = lov + RPV
            tcap = vt * CAP
            passbase = c * SCRH * 2 + h * SCRH
            myrow = s * RPV

            def init_acc(i, carry2):
                sqacc[pl.ds(i * 16, 16)] = z16
                mnacc[pl.ds(i * 16, 16)] = pinf
                mxacc[pl.ds(i * 16, 16)] = ninf
                return carry2

            _ = lax.fori_loop(0, RPV * D // 16, init_acc, 0)

            def zcopy(kk, carry2):
                pltpu.sync_copy(zbuf.at[pl.ds(0, 32)],
                                spm_sum.at[pl.ds(myrow + kk * 32, 32)])
                return carry2

            _ = lax.fori_loop(0, RPV // 32, zcopy, 0)

            @pl.when(s == 15)
            def _z():
                pltpu.sync_copy(zbuf.at[pl.ds(0, 1)],
                                spm_sum.at[pl.ds(SCRH, 1)])

            pltpu.sync_copy(nb_h.at[pl.ds(vt * 8, 8)], cntb.at[pl.ds(0, 8)])
            nb = cntb[pl.ds(0, 16)][0]

            def block_body(blk, carry2):
                pltpu.sync_copy(cd_h.at[pl.ds(tcap + blk * FB, FB)], cbd)
                pltpu.sync_copy(cs_h.at[pl.ds(tcap + blk * FB, FB)], cbs)
                pltpu.sync_copy(ce_h.at[pl.ds(tcap + blk * FB, FB)], cbe)

                def sub_body(sb, carry3):
                    sbase = sb * 64
                    for kk in range(4):
                        dv = cbd[pl.ds(sbase + kk * 16, 16)]
                        sv = cbs[pl.ds(sbase + kk * 16, 16)]
                        ev = cbe[pl.ds(sbase + kk * 16, 16)]
                        sstg[0, pl.ds(kk * 16, 16)] = jnp.minimum(sv, nmax)
                        dstg[0, pl.ds(kk * 16, 16)] = jnp.minimum(dv, nmax)
                        estg[0, pl.ds(kk * 16, 16)] = jnp.minimum(ev, emax)
                        lstg[0, pl.ds(kk * 16, 16)] = jnp.minimum(
                            dv - passbase, trash)
                    cpa = pltpu.make_async_copy(a_h.at[sstg.at[0]], bufa,
                                                semg)
                    cpb = pltpu.make_async_copy(b_h.at[dstg.at[0]], bufb,
                                                semg)
                    cpc = pltpu.make_async_copy(c_h.at[estg.at[0]], bufc,
                                                semg)
                    cpa.start()
                    cpb.start()
                    cpc.start()
                    cpa.wait()
                    cpb.wait()
                    cpc.wait()

                    def mrow(rr, carry4):
                        for col in range(0, D, 16):
                            mm = (bufa[rr, pl.ds(col, 16)]
                                  + bufb[rr, pl.ds(col, 16)]
                                  + bufc[rr, pl.ds(col, 16)])
                            bufa[rr, pl.ds(col, 16)] = mm
                        return carry4

                    _ = lax.fori_loop(0, 64, mrow, 0)

                    cps = pltpu.async_copy(bufa, spm_sum.at[lstg.at[0]],
                                           sems, add=True)

                    for g in range(4):
                        dv = cbd[pl.ds(sbase + g * 16, 16)]
                        for lane in range(16):
                            dg = dv[lane]

                            @pl.when(dg < hiv)
                            def _upd():
                                dl = dg - lov
                                rbase = dl * D
                                cols = list(range(0, D, 16))
                                row = g * 16 + lane
                                mms = [bufa[row, pl.ds(col, 16)]
                                       for col in cols]
                                sqv = [sqacc[pl.ds(rbase + col, 16)]
                                       for col in cols]
                                for i, col in enumerate(cols):
                                    sqacc[pl.ds(rbase + col, 16)] = \
                                        sqv[i] + mms[i] * mms[i]
                                mxv = [mxacc[pl.ds(rbase + col, 16)]
                                       for col in cols]
                                for i, col in enumerate(cols):
                                    mxacc[pl.ds(rbase + col, 16)] = \
                                        jnp.maximum(mxv[i], mms[i])
                                mnv = [mnacc[pl.ds(rbase + col, 16)]
                                       for col in cols]
                                for i, col in enumerate(cols):
                                    mnacc[pl.ds(rbase + col, 16)] = \
                                        jnp.minimum(mnv[i], mms[i])

                    cps.wait()
                    return carry3

                _ = lax.fori_loop(0, FB // 64, sub_body, 0)
                return carry2

            _ = lax.fori_loop(0, nb, block_body, 0)

            pltpu.sync_copy(spm_sum.at[pl.ds(myrow, RPV)],
                            sum_h.at[pl.ds(passbase + myrow, RPV)])
            pltpu.sync_copy(sqacc,
                            sq_h.at[pl.ds((passbase + myrow) * D, RPV * D)])
            pltpu.sync_copy(mnacc,
                            mn_h.at[pl.ds((passbase + myrow) * D, RPV * D)])
            pltpu.sync_copy(mxacc,
                            mx_h.at[pl.ds((passbase + myrow) * D, RPV * D)])
            return carry

        _ = lax.fori_loop(0, 2, pass_body, 0)

    return k(A, B, C, cd, cs, ce, nb_arr)


def _posttrans(h, mean, var, mn, mx, att, wq, bq):
    return (jnp.dot(h, wq[0:D, :], preferred_element_type=jnp.float32)
            + jnp.dot(mean, wq[D:2 * D, :],
                      preferred_element_type=jnp.float32)
            + jnp.dot(var, wq[2 * D:3 * D, :],
                      preferred_element_type=jnp.float32)
            + jnp.dot(mn, wq[3 * D:4 * D, :],
                      preferred_element_type=jnp.float32)
            + jnp.dot(mx, wq[4 * D:5 * D, :],
                      preferred_element_type=jnp.float32)
            + jnp.dot(mean * att, wq[5 * D:6 * D, :],
                      preferred_element_type=jnp.float32)
            + jnp.dot(var * att, wq[6 * D:7 * D, :],
                      preferred_element_type=jnp.float32)
            + jnp.dot(mn * att, wq[7 * D:8 * D, :],
                      preferred_element_type=jnp.float32)
            + jnp.dot(mx * att, wq[8 * D:9 * D, :],
                      preferred_element_type=jnp.float32)
            + bq)


def _aggs(su, sq, mns, mxs, deg):
    degc = jnp.maximum(deg, 1.0)
    inv = 1.0 / degc
    mean = su * inv
    var = jnp.maximum(sq * inv - mean * mean, 0.0)
    pos = deg > 0
    mn = jnp.where(pos, mns, 0.0)
    mx = jnp.where(pos, mxs, 0.0)
    att = DELTA / jnp.log(degc + 1.0)
    return mean, var, mn, mx, att


def _node_update(h, sums, sqs, mns, mxs, degs, W_post, b_post, W_pre_next):
    BN = 1000

    def body(h_ref, su_ref, sq_ref, mn_ref, mx_ref, dg_ref, wq_ref, bq_ref,
             wp_ref, h1_ref, a_ref, b_ref):
        mean, var, mn, mx, att = _aggs(su_ref[...], sq_ref[...], mn_ref[...],
                                       mx_ref[...], dg_ref[...])
        hn = _posttrans(h_ref[...], mean, var, mn, mx, att, wq_ref[...],
                        bq_ref[...])
        h1_ref[...] = hn
        a_ref[...] = jnp.dot(hn, wp_ref[0:D, :],
                             preferred_element_type=jnp.float32)
        b_ref[...] = jnp.dot(hn, wp_ref[D:2 * D, :],
                             preferred_element_type=jnp.float32)

    return pl.pallas_call(
        body,
        grid=(N // BN,),
        in_specs=[
            pl.BlockSpec((BN, D), lambda i: (i, 0)),
            pl.BlockSpec((BN, D), lambda i: (i, 0)),
            pl.BlockSpec((BN, D), lambda i: (i, 0)),
            pl.BlockSpec((BN, D), lambda i: (i, 0)),
            pl.BlockSpec((BN, D), lambda i: (i, 0)),
            pl.BlockSpec((BN, 1), lambda i: (i, 0)),
            pl.BlockSpec((9 * D, D), lambda i: (0, 0)),
            pl.BlockSpec((1, D), lambda i: (0, 0)),
            pl.BlockSpec((3 * D, D), lambda i: (0, 0)),
        ],
        out_specs=[
            pl.BlockSpec((BN, D), lambda i: (i, 0)),
            pl.BlockSpec((BN, D), lambda i: (i, 0)),
            pl.BlockSpec((BN, D), lambda i: (i, 0)),
        ],
        out_shape=[
            jax.ShapeDtypeStruct((N, D), jnp.float32),
            jax.ShapeDtypeStruct((N, D), jnp.float32),
            jax.ShapeDtypeStruct((N, D), jnp.float32),
        ],
    )(h, sums, sqs, mns, mxs, degs, W_post, b_post.reshape(1, D), W_pre_next)


def _node_update_final(h, sums, sqs, mns, mxs, degs, W_post, b_post,
                       gid3, W_out, b_out):
    BN = 1000
    NB = N // BN

    def body(h_ref, su_ref, sq_ref, mn_ref, mx_ref, dg_ref, wq_ref, bq_ref,
             gid_ref, wo_ref, bo_ref, out_ref, pool_ref, cnt_ref):
        i = pl.program_id(0)

        @pl.when(i == 0)
        def _init():
            pool_ref[...] = jnp.zeros((NG, D), jnp.float32)
            cnt_ref[...] = jnp.zeros((NG, D), jnp.float32)

        mean, var, mn, mx, att = _aggs(su_ref[...], sq_ref[...], mn_ref[...],
                                       mx_ref[...], dg_ref[...])
        hn = _posttrans(h_ref[...], mean, var, mn, mx, att, wq_ref[...],
                        bq_ref[...])
        onehot = jnp.where(
            gid_ref[0] == lax.broadcasted_iota(jnp.int32, (NG, BN), 0),
            1.0, 0.0).astype(jnp.float32)
        pool_ref[...] += jnp.dot(onehot, hn,
                                 preferred_element_type=jnp.float32)
        cnt_ref[...] += jnp.dot(onehot, jnp.ones((BN, D), jnp.float32),
                                preferred_element_type=jnp.float32)

        @pl.when(i == NB - 1)
        def _fin():
            cnt = jnp.maximum(cnt_ref[...], 1.0)
            pooled = pool_ref[...] / cnt
            out_ref[...] = jnp.dot(pooled, wo_ref[...],
                                   preferred_element_type=jnp.float32) \
                + bo_ref[...]

    return pl.pallas_call(
        body,
        grid=(NB,),
        in_specs=[
            pl.BlockSpec((BN, D), lambda i: (i, 0)),
            pl.BlockSpec((BN, D), lambda i: (i, 0)),
            pl.BlockSpec((BN, D), lambda i: (i, 0)),
            pl.BlockSpec((BN, D), lambda i: (i, 0)),
            pl.BlockSpec((BN, D), lambda i: (i, 0)),
            pl.BlockSpec((BN, 1), lambda i: (i, 0)),
            pl.BlockSpec((9 * D, D), lambda i: (0, 0)),
            pl.BlockSpec((1, D), lambda i: (0, 0)),
            pl.BlockSpec((1, 1, BN), lambda i: (i, 0, 0)),
            pl.BlockSpec((D, 1), lambda i: (0, 0)),
            pl.BlockSpec((1, 1), lambda i: (0, 0)),
        ],
        out_specs=pl.BlockSpec((NG, 1), lambda i: (0, 0)),
        out_shape=jax.ShapeDtypeStruct((NG, 1), jnp.float32),
        scratch_shapes=[
            pltpu.VMEM((NG, D), jnp.float32),
            pltpu.VMEM((NG, D), jnp.float32),
        ],
    )(h, sums, sqs, mns, mxs, degs, W_post, b_post.reshape(1, D), gid3,
      W_out, b_out.reshape(1, 1))


def kernel(edge_index, r, atom_features, distances, graph_ids, af_table,
           W_atom, b_atom, W_dist, b_dist, ln_g, ln_b, W_edge, b_edge,
           W_pre0, b_pre0, W_post0, b_post0, W_pre1, b_pre1, W_post1, b_post1,
           W_out, b_out):
    src = edge_index[0].astype(jnp.int32)
    dst = edge_index[1].astype(jnp.int32)
    src_p = jnp.pad(src, (0, EPADP - E))
    dst_p = jnp.pad(dst, (0, EPADP - E))
    af2d = atom_features.astype(jnp.int32).reshape(N, 1)
    gid3 = graph_ids.astype(jnp.int32).reshape(N // 1000, 1, 1000)

    table2, wc0, cb0, wc1, cb1 = _prep_weights(
        af_table, W_atom, b_atom, W_edge, b_edge, W_pre0, b_pre0,
        W_pre1, b_pre1)
    h0, a0, b0 = _embed(af2d, distances, table2, W_dist, b_dist, ln_g, ln_b,
                        W_pre0)
    c0, c1 = _edge_c(r, wc0, cb0, wc1, cb1)
    cd, cs, ce, nb_arr, degflat = _sc_prep(dst_p, src_p)
    degs = degflat.reshape(NPAD, 16)[:N, 0:1]

    sums0, sqf0, mnf0, mxf0 = _sc_layer(a0, b0, c0, cd, cs, ce, nb_arr)
    h1, a1, b1 = _node_update(
        h0, sums0[:N], sqf0.reshape(NPAD, D)[:N],
        mnf0.reshape(NPAD, D)[:N], mxf0.reshape(NPAD, D)[:N],
        degs, W_post0, b_post0, W_pre1)

    sums1, sqf1, mnf1, mxf1 = _sc_layer(a1, b1, c1, cd, cs, ce, nb_arr)
    out = _node_update_final(
        h1, sums1[:N], sqf1.reshape(NPAD, D)[:N],
        mnf1.reshape(NPAD, D)[:N], mxf1.reshape(NPAD, D)[:N],
        degs, W_post1, b_post1, gid3, W_out, b_out)
    return out
